# baseline (speedup 1.0000x reference)
import functools

import jax
import jax.numpy as jnp
from jax import lax
from jax.experimental import pallas as pl
from jax.experimental.pallas import tpu as pltpu
from jax.experimental.pallas import tpu_sc as plsc

_BATCH = 16384
_HIST = 50
_DIM = 64
_LANES = 16
_NBINS = 512
_QS = _NBINS // 128
_NCORES = 2
_NSUB = 16
_NW = _NCORES * _NSUB
_CHUNK = 64
_GPC = _CHUNK // _LANES
_NSPLIT = 1


def _sc_counts(x_flat, lut, nb, off):
    bpw = nb // _NW
    nchunk = bpw // _CHUNK
    mesh = plsc.VectorSubcoreMesh(
        core_axis_name="c", subcore_axis_name="s",
        num_cores=_NCORES, num_subcores=_NSUB)

    @functools.partial(
        pl.kernel,
        out_type=jax.ShapeDtypeStruct((_QS * nb * 128,), jnp.float32),
        mesh=mesh,
        scratch_types=[
            pltpu.VMEM((_CHUNK, _HIST), jnp.int32),
            pltpu.VMEM((_CHUNK, _HIST), jnp.int32),
            pltpu.VMEM((_NBINS,), jnp.int32),
            pltpu.VMEM((_CHUNK * _NBINS,), jnp.float32),
            pltpu.VMEM((_CHUNK * _NBINS,), jnp.float32),
            pltpu.SemaphoreType.DMA,
            pltpu.SemaphoreType.DMA,
            pltpu.SemaphoreType.DMA,
        ],
        compiler_params=pltpu.CompilerParams(needs_layout_passes=False),
    )
    def k(x_hbm, lut_hbm, counts_hbm, xv0, xv1, lutv, hist0, hist1,
          sem0, sem1, xsem):
        wid = lax.axis_index("s") * _NCORES + lax.axis_index("c")
        pltpu.sync_copy(lut_hbm, lutv)
        xvs = (xv0, xv1)
        xpend = pltpu.async_copy(
            x_hbm.at[pl.ds(off + wid * bpw, _CHUNK)], xv0, xsem)

        lanes = lax.broadcasted_iota(jnp.int32, (_LANES,), 0)
        zeros16 = jnp.zeros((_LANES,), jnp.float32)
        ones16 = jnp.ones((_LANES,), jnp.float32)
        hists = (hist0, hist1)
        sems = (sem0, sem1)
        pending = [None, None]

        for c in range(nchunk):
            hist = hists[c % 2]
            xv = xvs[c % 2]
            xpend.wait()
            if c + 1 < nchunk:
                xpend = pltpu.async_copy(
                    x_hbm.at[pl.ds(off + wid * bpw + (c + 1) * _CHUNK, _CHUNK)],
                    xvs[(c + 1) % 2], xsem)
            if pending[c % 2] is not None:
                for p in pending[c % 2]:
                    p.wait()

            _ZUNROLL = 32
            def zero_body(i, _):
                for z in range(_ZUNROLL):
                    hist[pl.ds((i * _ZUNROLL + z) * _LANES, _LANES)] = zeros16
                return _
            lax.fori_loop(0, _CHUNK * _NBINS // (_LANES * _ZUNROLL),
                          zero_body, None)

            @plsc.parallel_loop(0, _HIST * _GPC, step=1, unroll=8)
            def _scatter(i):
                g = i & (_GPC - 1)
                l = lax.shift_right_logical(i, 2)
                bag_local = lax.shift_left(g, 4) + lanes
                idx = plsc.load_gather(
                    xv, [bag_local, jnp.full((_LANES,), l, jnp.int32)])
                row = plsc.load_gather(lutv, [idx])
                q = lax.shift_right_logical(row, 7)
                r = row & 127
                pos = (lax.shift_left(q, 13) + lax.shift_left(bag_local, 7)) + r
                plsc.addupdate_scatter(hist, [pos], ones16)

            bagbase = wid * bpw + c * _CHUNK
            ps = []
            for q in range(_QS):
                dst = counts_hbm.at[pl.ds(
                    (q * nb + bagbase) * 128, _CHUNK * 128)]
                src = hist.at[pl.ds(q * _CHUNK * 128, _CHUNK * 128)]
                ps.append(pltpu.async_copy(src, dst, sems[c % 2]))
            pending[c % 2] = ps

        for p in pending[0] + pending[1]:
            p.wait()

    return k(x_flat, lut)


def _tc_matmul(counts3, w3, nb):
    bb = 2048

    def body(c_ref, w_ref, o_ref):
        acc = jnp.dot(c_ref[0], w_ref[0], preferred_element_type=jnp.float32)
        for q in range(1, _QS):
            acc += jnp.dot(c_ref[q], w_ref[q],
                           preferred_element_type=jnp.float32)
        o_ref[...] = acc

    return pl.pallas_call(
        body,
        grid=(nb // bb,),
        in_specs=[
            pl.BlockSpec((_QS, bb, 128), lambda i: (0, i, 0)),
            pl.BlockSpec((_QS, 128, _DIM), lambda i: (0, 0, 0)),
        ],
        out_specs=pl.BlockSpec((bb, _DIM), lambda i: (i, 0)),
        out_shape=jax.ShapeDtypeStruct((nb, _DIM), jnp.float32),
    )(counts3, w3)


def kernel(x, weight):
    r = jnp.arange(_NBINS, dtype=jnp.int32)
    lut = jnp.clip(
        r.astype(jnp.float32).astype(jnp.float8_e4m3fn).astype(jnp.float32)
        .astype(jnp.int32), 0, _NBINS - 1)
    w3 = weight[:_NBINS].reshape(_QS, 128, _DIM)
    nb = _BATCH // _NSPLIT
    outs = []
    for h in range(_NSPLIT):
        counts = _sc_counts(x, lut, nb, h * nb)
        outs.append(_tc_matmul(counts.reshape(_QS, nb, 128), w3, nb))
    return outs[0] if _NSPLIT == 1 else jnp.concatenate(outs, axis=0)


# speedup vs baseline: 1.0064x; 1.0064x over previous
import functools

import jax
import jax.numpy as jnp
from jax import lax
from jax.experimental import pallas as pl
from jax.experimental.pallas import tpu as pltpu
from jax.experimental.pallas import tpu_sc as plsc

_BATCH = 16384
_HIST = 50
_DIM = 64
_LANES = 16
_NBINS = 512
_QS = _NBINS // 128
_NCORES = 2
_NSUB = 16
_NW = _NCORES * _NSUB
_CHUNK = 64
_GPC = _CHUNK // _LANES
_NSPLIT = 1


def _sc_counts(x_flat, lut, nb, off):
    bpw = nb // _NW
    nchunk = bpw // _CHUNK
    mesh = plsc.VectorSubcoreMesh(
        core_axis_name="c", subcore_axis_name="s",
        num_cores=_NCORES, num_subcores=_NSUB)

    @functools.partial(
        pl.kernel,
        out_type=jax.ShapeDtypeStruct((_QS * nb * 128,), jnp.float32),
        mesh=mesh,
        scratch_types=[
            pltpu.VMEM((_CHUNK, _HIST), jnp.int32),
            pltpu.VMEM((_CHUNK, _HIST), jnp.int32),
            pltpu.VMEM((_NBINS,), jnp.int32),
            pltpu.VMEM((_CHUNK * _NBINS,), jnp.float32),
            pltpu.VMEM((_CHUNK * _NBINS,), jnp.float32),
            pltpu.SemaphoreType.DMA,
            pltpu.SemaphoreType.DMA,
            pltpu.SemaphoreType.DMA,
        ],
        compiler_params=pltpu.CompilerParams(needs_layout_passes=False),
    )
    def k(x_hbm, lut_hbm, counts_hbm, xv0, xv1, lutv, hist0, hist1,
          sem0, sem1, xsem):
        wid = lax.axis_index("s") * _NCORES + lax.axis_index("c")
        pltpu.sync_copy(lut_hbm, lutv)
        xvs = (xv0, xv1)
        xpend = pltpu.async_copy(
            x_hbm.at[pl.ds(off + wid * bpw, _CHUNK)], xv0, xsem)

        lanes = lax.broadcasted_iota(jnp.int32, (_LANES,), 0)
        zeros16 = jnp.zeros((_LANES,), jnp.float32)
        ones16 = jnp.ones((_LANES,), jnp.float32)
        hists = (hist0, hist1)
        sems = (sem0, sem1)
        pending = [None, None]

        for c in range(nchunk):
            hist = hists[c % 2]
            xv = xvs[c % 2]
            xpend.wait()
            if c + 1 < nchunk:
                xpend = pltpu.async_copy(
                    x_hbm.at[pl.ds(off + wid * bpw + (c + 1) * _CHUNK, _CHUNK)],
                    xvs[(c + 1) % 2], xsem)
            if pending[c % 2] is not None:
                for p in pending[c % 2]:
                    p.wait()

            _ZUNROLL = 32
            def zero_body(i, _):
                for z in range(_ZUNROLL):
                    hist[pl.ds((i * _ZUNROLL + z) * _LANES, _LANES)] = zeros16
                return _
            lax.fori_loop(0, _CHUNK * _NBINS // (_LANES * _ZUNROLL),
                          zero_body, None)

            @plsc.parallel_loop(0, _HIST * _GPC, step=1, unroll=4)
            def _scatter(i):
                g = i & (_GPC - 1)
                l = lax.shift_right_logical(i, 2)
                bag_local = lax.shift_left(g, 4) + lanes
                idx = plsc.load_gather(
                    xv, [bag_local, jnp.full((_LANES,), l, jnp.int32)])
                row = plsc.load_gather(lutv, [idx])
                q = lax.shift_right_logical(row, 7)
                r = row & 127
                pos = (lax.shift_left(q, 13) + lax.shift_left(bag_local, 7)) + r
                plsc.addupdate_scatter(hist, [pos], ones16)

            bagbase = wid * bpw + c * _CHUNK
            ps = []
            for q in range(_QS):
                dst = counts_hbm.at[pl.ds(
                    (q * nb + bagbase) * 128, _CHUNK * 128)]
                src = hist.at[pl.ds(q * _CHUNK * 128, _CHUNK * 128)]
                ps.append(pltpu.async_copy(src, dst, sems[c % 2]))
            pending[c % 2] = ps

        for p in pending[0] + pending[1]:
            p.wait()

    return k(x_flat, lut)


def _tc_matmul(counts3, w3, nb):
    bb = 2048

    def body(c_ref, w_ref, o_ref):
        acc = jnp.dot(c_ref[0], w_ref[0], preferred_element_type=jnp.float32)
        for q in range(1, _QS):
            acc += jnp.dot(c_ref[q], w_ref[q],
                           preferred_element_type=jnp.float32)
        o_ref[...] = acc

    return pl.pallas_call(
        body,
        grid=(nb // bb,),
        in_specs=[
            pl.BlockSpec((_QS, bb, 128), lambda i: (0, i, 0)),
            pl.BlockSpec((_QS, 128, _DIM), lambda i: (0, 0, 0)),
        ],
        out_specs=pl.BlockSpec((bb, _DIM), lambda i: (i, 0)),
        out_shape=jax.ShapeDtypeStruct((nb, _DIM), jnp.float32),
    )(counts3, w3)


def kernel(x, weight):
    r = jnp.arange(_NBINS, dtype=jnp.int32)
    lut = jnp.clip(
        r.astype(jnp.float32).astype(jnp.float8_e4m3fn).astype(jnp.float32)
        .astype(jnp.int32), 0, _NBINS - 1)
    w3 = weight[:_NBINS].reshape(_QS, 128, _DIM)
    nb = _BATCH // _NSPLIT
    outs = []
    for h in range(_NSPLIT):
        counts = _sc_counts(x, lut, nb, h * nb)
        outs.append(_tc_matmul(counts.reshape(_QS, nb, 128), w3, nb))
    return outs[0] if _NSPLIT == 1 else jnp.concatenate(outs, axis=0)
